# baseline (speedup 1.0000x reference)
import functools

import jax
import jax.numpy as jnp
from jax import lax
from jax.experimental import pallas as pl
from jax.experimental.pallas import tpu as pltpu
from jax.experimental.pallas import tpu_sc as plsc

EMBED_DIM = 64
PAD_DIM = 128
CHUNK = 128
NBUF = 4


@functools.cache
def _make_kernel(n_out: int):
    info = plsc.get_sparse_core_info()
    num_cores = info.num_cores
    num_workers = info.num_cores * info.num_subcores
    b_per_w = n_out // num_workers
    assert n_out % num_workers == 0 and b_per_w % CHUNK == 0
    n_chunks = b_per_w // CHUNK
    n_groups = n_chunks // NBUF
    assert n_chunks % NBUF == 0 and n_groups >= 3

    mesh = plsc.VectorSubcoreMesh(core_axis_name="c", subcore_axis_name="s")

    @functools.partial(
        pl.kernel,
        mesh=mesh,
        out_type=jax.ShapeDtypeStruct((n_out, PAD_DIM), jnp.float32),
        scratch_types=[
            pltpu.VMEM((b_per_w,), jnp.int32),
            pltpu.VMEM((b_per_w,), jnp.int32),
            pltpu.VMEM((NBUF, CHUNK, PAD_DIM), jnp.float32),
            pltpu.SemaphoreType.DMA,
            pltpu.SemaphoreType.DMA,
            pltpu.SemaphoreType.DMA,
        ],
    )
    def emb_kernel(idx_e_hbm, idx_o_hbm, tab_l_hbm, tab_r_hbm, out_hbm,
                   idx_e_v, idx_o_v, buf, gl_sem, gr_sem, o_sem):
        wid = lax.axis_index("s") * num_cores + lax.axis_index("c")
        base = wid * b_per_w
        pltpu.sync_copy(idx_e_hbm.at[pl.ds(base, b_per_w)], idx_e_v)
        pltpu.sync_copy(idx_o_hbm.at[pl.ds(base, b_per_w)], idx_o_v)

        def start_gl(i, b):
            off = pl.multiple_of(i * CHUNK, CHUNK)
            pltpu.async_copy(
                tab_l_hbm.at[idx_e_v.at[pl.ds(off, CHUNK)]], buf.at[b],
                gl_sem)

        def wait_gl(b):
            pltpu.make_async_copy(
                tab_l_hbm.at[idx_e_v.at[pl.ds(0, CHUNK)]], buf.at[b],
                gl_sem).wait()

        def start_gr(i, b):
            off = pl.multiple_of(i * CHUNK, CHUNK)
            pltpu.async_copy(
                tab_r_hbm.at[idx_o_v.at[pl.ds(off, CHUNK)]], buf.at[b],
                gr_sem, add=True)

        def wait_gr(b):
            pltpu.make_async_copy(
                tab_r_hbm.at[idx_o_v.at[pl.ds(0, CHUNK)]], buf.at[b],
                gr_sem).wait()

        def start_out(i, b):
            off = pl.multiple_of(base + i * CHUNK, CHUNK)
            pltpu.async_copy(buf.at[b], out_hbm.at[pl.ds(off, CHUNK)], o_sem)

        def wait_out(b):
            pltpu.make_async_copy(
                buf.at[b], out_hbm.at[pl.ds(0, CHUNK)], o_sem).wait()

        def visit(i, b, retire_prev=True, start_next=True):
            wait_gl(b)
            start_gr(i, b)
            if retire_prev:
                wait_out((b - 1) % NBUF)
            if start_next:
                start_gl(i - 1 + NBUF, (b - 1) % NBUF)
            wait_gr(b)
            start_out(i, b)

        for b in range(NBUF):
            start_gl(b, b)

        for b in range(NBUF):
            visit(b, b, retire_prev=b >= 1, start_next=b >= 1)

        @pl.loop(1, n_groups - 1)
        def _(t):
            for b in range(NBUF):
                visit(t * NBUF + b, b, start_next=True)

        for b in range(NBUF):
            i = (n_groups - 1) * NBUF + b
            visit(i, b, start_next=(i - 1 + NBUF) < n_chunks)

        wait_out((n_chunks - 1) % NBUF)

    return emb_kernel


@jax.jit
def kernel(input_id, table):
    batch, seq_len = input_id.shape
    n_idx = batch * seq_len
    n_out = n_idx // 2
    idx_pair_t = input_id.reshape(n_out, 2).T
    tab_left = jnp.pad(table, ((0, 0), (0, PAD_DIM - EMBED_DIM)))
    tab_right = jnp.pad(table, ((0, 0), (PAD_DIM - EMBED_DIM, 0)))
    out = _make_kernel(n_out)(idx_pair_t[0], idx_pair_t[1],
                              tab_left, tab_right)
    return out.reshape(batch, seq_len, EMBED_DIM)


# speedup vs baseline: 1.3123x; 1.3123x over previous
import functools

import jax
import jax.numpy as jnp
from jax import lax
from jax.experimental import pallas as pl
from jax.experimental.pallas import tpu as pltpu
from jax.experimental.pallas import tpu_sc as plsc

EMBED_DIM = 64
PAD_DIM = 128
CHUNK = 128
NBUF = 4


@functools.cache
def _make_kernel(n_out: int):
    info = plsc.get_sparse_core_info()
    num_cores = info.num_cores
    num_workers = info.num_cores * info.num_subcores
    b_per_w = n_out // num_workers
    assert n_out % num_workers == 0 and b_per_w % CHUNK == 0
    n_chunks = b_per_w // CHUNK
    n_groups = n_chunks // NBUF
    assert n_chunks % NBUF == 0 and n_groups >= 3

    mesh = plsc.VectorSubcoreMesh(core_axis_name="c", subcore_axis_name="s")

    @functools.partial(
        pl.kernel,
        mesh=mesh,
        out_type=jax.ShapeDtypeStruct((n_out, PAD_DIM), jnp.float32),
        scratch_types=[
            pltpu.VMEM((2 * b_per_w,), jnp.int32),
            pltpu.VMEM((b_per_w,), jnp.int32),
            pltpu.VMEM((b_per_w,), jnp.int32),
            pltpu.VMEM((NBUF, CHUNK, PAD_DIM), jnp.float32),
            pltpu.SemaphoreType.DMA,
            pltpu.SemaphoreType.DMA,
            pltpu.SemaphoreType.DMA,
        ],
    )
    def emb_kernel(idx_hbm, tab_l_hbm, tab_r_hbm, out_hbm,
                   idx_v, idx_e_v, idx_o_v, buf, gl_sem, gr_sem, o_sem):
        wid = lax.axis_index("s") * num_cores + lax.axis_index("c")
        base = wid * b_per_w
        pltpu.sync_copy(idx_hbm.at[pl.ds(2 * base, 2 * b_per_w)], idx_v)

        lanes = lax.iota(jnp.int32, 16)
        perm_e = (2 * lanes) % 16
        lane_lo = lanes < 8

        def shuffle(va, vb, perm):
            ga = va.at[perm].get(mode="promise_in_bounds")
            gb = vb.at[perm].get(mode="promise_in_bounds")
            return jnp.where(lane_lo, ga, gb)

        def deint(i):
            off = i * CHUNK
            for t in range(CHUNK // 16):
                o2 = 2 * off + 32 * t
                va = idx_v[pl.ds(o2, 16)]
                vb = idx_v[pl.ds(o2 + 16, 16)]
                idx_e_v[pl.ds(off + 16 * t, 16)] = shuffle(va, vb, perm_e)
                idx_o_v[pl.ds(off + 16 * t, 16)] = shuffle(va, vb, perm_e + 1)

        def start_gl(i, b):
            off = pl.multiple_of(i * CHUNK, CHUNK)
            pltpu.async_copy(
                tab_l_hbm.at[idx_e_v.at[pl.ds(off, CHUNK)]], buf.at[b],
                gl_sem)

        def wait_gl(b):
            pltpu.make_async_copy(
                tab_l_hbm.at[idx_e_v.at[pl.ds(0, CHUNK)]], buf.at[b],
                gl_sem).wait()

        def start_gr(i, b):
            off = pl.multiple_of(i * CHUNK, CHUNK)
            pltpu.async_copy(
                tab_r_hbm.at[idx_o_v.at[pl.ds(off, CHUNK)]], buf.at[b],
                gr_sem, add=True)

        def wait_gr(b):
            pltpu.make_async_copy(
                tab_r_hbm.at[idx_o_v.at[pl.ds(0, CHUNK)]], buf.at[b],
                gr_sem).wait()

        def start_out(i, b):
            off = pl.multiple_of(base + i * CHUNK, CHUNK)
            pltpu.async_copy(buf.at[b], out_hbm.at[pl.ds(off, CHUNK)], o_sem)

        def wait_out(b):
            pltpu.make_async_copy(
                buf.at[b], out_hbm.at[pl.ds(0, CHUNK)], o_sem).wait()

        def visit(i, b, retire_prev=True, start_next=True):
            wait_gl(b)
            start_gr(i, b)
            if retire_prev:
                wait_out((b - 1) % NBUF)
            if start_next:
                deint(i - 1 + NBUF)
                start_gl(i - 1 + NBUF, (b - 1) % NBUF)
            wait_gr(b)
            start_out(i, b)

        for b in range(NBUF):
            deint(b)
            start_gl(b, b)

        for b in range(NBUF):
            visit(b, b, retire_prev=b >= 1, start_next=b >= 1)

        @pl.loop(1, n_groups - 1)
        def _(t):
            for b in range(NBUF):
                visit(t * NBUF + b, b, start_next=True)

        for b in range(NBUF):
            i = (n_groups - 1) * NBUF + b
            visit(i, b, start_next=(i - 1 + NBUF) < n_chunks)

        wait_out((n_chunks - 1) % NBUF)

    return emb_kernel


@jax.jit
def kernel(input_id, table):
    batch, seq_len = input_id.shape
    n_idx = batch * seq_len
    n_out = n_idx // 2
    flat_idx = input_id.reshape(n_idx)
    tab_left = jnp.pad(table, ((0, 0), (0, PAD_DIM - EMBED_DIM)))
    tab_right = jnp.pad(table, ((0, 0), (PAD_DIM - EMBED_DIM, 0)))
    out = _make_kernel(n_out)(flat_idx, tab_left, tab_right)
    return out.reshape(batch, seq_len, EMBED_DIM)


# speedup vs baseline: 1.9731x; 1.5036x over previous
import functools

import jax
import jax.numpy as jnp
from jax import lax
from jax.experimental import pallas as pl
from jax.experimental.pallas import tpu as pltpu
from jax.experimental.pallas import tpu_sc as plsc

EMBED_DIM = 64
PAD_DIM = 128
CHUNK = 256
NBUF = 2


@functools.cache
def _make_kernel(n_idx: int):
    info = plsc.get_sparse_core_info()
    num_cores = info.num_cores
    num_workers = info.num_cores * info.num_subcores
    b_per_w = n_idx // num_workers
    assert n_idx % num_workers == 0 and b_per_w % CHUNK == 0
    n_chunks = b_per_w // CHUNK
    n_groups = n_chunks // NBUF
    assert n_chunks % NBUF == 0 and n_groups >= 3

    mesh = plsc.VectorSubcoreMesh(core_axis_name="c", subcore_axis_name="s")

    @functools.partial(
        pl.kernel,
        mesh=mesh,
        out_type=jax.ShapeDtypeStruct((n_idx, PAD_DIM), jnp.float32),
        scratch_types=[
            pltpu.VMEM((b_per_w,), jnp.int32),
            pltpu.VMEM((NBUF, CHUNK, PAD_DIM), jnp.float32),
            pltpu.SemaphoreType.DMA,
            pltpu.SemaphoreType.DMA,
        ],
    )
    def emb_kernel(idx_hbm, table_hbm, out_hbm, idx_v, rows_v, gsem, osem):
        wid = lax.axis_index("s") * num_cores + lax.axis_index("c")
        base = wid * b_per_w
        pltpu.sync_copy(idx_hbm.at[pl.ds(base, b_per_w)], idx_v)

        def start_gather(i, b):
            off = pl.multiple_of(i * CHUNK, CHUNK)
            pltpu.async_copy(
                table_hbm.at[idx_v.at[pl.ds(off, CHUNK)]], rows_v.at[b], gsem)

        def wait_gather(b):
            pltpu.make_async_copy(
                table_hbm.at[idx_v.at[pl.ds(0, CHUNK)]], rows_v.at[b], gsem
            ).wait()

        def start_out(i, b):
            off = pl.multiple_of(base + i * CHUNK, CHUNK)
            pltpu.async_copy(rows_v.at[b], out_hbm.at[pl.ds(off, CHUNK)], osem)

        def wait_out(b):
            pltpu.make_async_copy(
                rows_v.at[b], out_hbm.at[pl.ds(0, CHUNK)], osem
            ).wait()

        def visit(i, b, retire_prev=True, start_next=True):
            wait_gather(b)
            start_out(i, b)
            if retire_prev:
                wait_out((b - 1) % NBUF)
            if start_next:
                start_gather(i - 1 + NBUF, (b - 1) % NBUF)

        for b in range(NBUF):
            start_gather(b, b)

        for b in range(NBUF):
            visit(b, b, retire_prev=b >= 1, start_next=b >= 1)

        @pl.loop(1, n_groups - 1)
        def _(t):
            for b in range(NBUF):
                visit(t * NBUF + b, b, start_next=True)

        for b in range(NBUF):
            i = (n_groups - 1) * NBUF + b
            visit(i, b, start_next=(i - 1 + NBUF) < n_chunks)

        wait_out((n_chunks - 1) % NBUF)

    return emb_kernel


@jax.jit
def kernel(input_id, table):
    batch, seq_len = input_id.shape
    flat_idx = input_id.reshape(batch * seq_len)
    padded = jnp.pad(table, ((0, 0), (0, PAD_DIM - EMBED_DIM)))
    out = _make_kernel(batch * seq_len)(flat_idx, padded)
    return out[:, :EMBED_DIM].reshape(batch, seq_len, EMBED_DIM)
